# baseline (speedup 1.0000x reference)
import functools

import jax
import jax.numpy as jnp
from jax import lax
from jax.experimental import pallas as pl
from jax.experimental.pallas import tpu as pltpu
from jax.experimental.pallas import tpu_sc as plsc

_NC = 2
_NS = 16
_NW = _NC * _NS
_CHUNK = 128


def _sc_gather(feat2d, aux2d, idx):
    rows, cf_w = feat2d.shape
    aux_w = aux2d.shape[1]
    per_w = rows // _NW
    n_chunks = per_w // _CHUNK
    mesh = plsc.VectorSubcoreMesh(core_axis_name="c", subcore_axis_name="s")

    @functools.partial(
        pl.kernel,
        mesh=mesh,
        out_type=(
            jax.ShapeDtypeStruct((rows, cf_w), jnp.int32),
            jax.ShapeDtypeStruct((rows, aux_w), jnp.float32),
        ),
        scratch_types=[
            pltpu.VMEM((_CHUNK,), jnp.int32),
            pltpu.VMEM((_CHUNK, cf_w), jnp.int32),
            pltpu.VMEM((_CHUNK, aux_w), jnp.float32),
            pltpu.SemaphoreType.DMA,
            pltpu.SemaphoreType.DMA,
        ],
    )
    def gather_kernel(feat_hbm, aux_hbm, idx_hbm, gf_hbm, ga_hbm,
                      idx_v, frows_v, arows_v, sem_f, sem_a):
        wid = lax.axis_index("s") * _NC + lax.axis_index("c")
        base = wid * per_w
        for ci in range(n_chunks):
            off = base + ci * _CHUNK
            pltpu.sync_copy(idx_hbm.at[pl.ds(off, _CHUNK)], idx_v)
            cp_f = pltpu.async_copy(feat_hbm.at[idx_v], frows_v, sem_f)
            cp_a = pltpu.async_copy(aux_hbm.at[idx_v], arows_v, sem_a)
            cp_f.wait()
            cp_a.wait()
            pltpu.sync_copy(frows_v, gf_hbm.at[pl.ds(off, _CHUNK)])
            pltpu.sync_copy(arows_v, ga_hbm.at[pl.ds(off, _CHUNK)])

    return gather_kernel(feat2d, aux2d, idx)


def _attn_body(c, c_, h, d, m, zb, scale,
               gf_ref, ga_ref, wqkv_ref, bqkv_ref, wpos_ref, wproj_ref,
               bproj_ref, out_ref):
    r = zb * m
    qkv = jnp.dot(gf_ref[...], wqkv_ref[...],
                  preferred_element_type=jnp.float32) + bqkv_ref[...]
    aux_t = ga_ref[...].T
    cf = ga_ref[...][:, 0:c_]
    cf_t = aux_t[0:c_, :]
    pos_t = aux_t[c_:c_ + d, :]
    mask_t = aux_t[c_ + d:c_ + d + 1, :]
    sq_t = jnp.sum(cf_t * cf_t, axis=0, keepdims=True)
    pb_t = jnp.dot(wpos_ref[...], pos_t, precision=lax.Precision.HIGHEST,
                   preferred_element_type=jnp.float32)
    row_c = lax.broadcasted_iota(jnp.int32, (r, r), 0)
    col_c = lax.broadcasted_iota(jnp.int32, (r, r), 1)
    blockpen = jnp.where((row_c // m) == (col_c // m), 0.0, -1e30)
    cfdot = lax.dot_general(cf, cf, (((1,), (1,)), ((), ())),
                            precision=lax.Precision.HIGHEST,
                            preferred_element_type=jnp.float32)
    base = 2.0 * cfdot + blockpen + (-sq_t - 100.0 * (1.0 - mask_t))
    ls = []
    for hh in range(h):
        qh = qkv[:, hh * c_:(hh + 1) * c_] * scale
        kh = qkv[:, c + hh * c_:c + (hh + 1) * c_]
        l = lax.dot_general(qh, kh, (((1,), (1,)), ((), ())),
                            preferred_element_type=jnp.float32)
        ls.append(l + base + pb_t[hh:hh + 1, :])
    es, ss = [], []
    for hh in range(h):
        mx = jnp.max(ls[hh], axis=1, keepdims=True)
        e = jnp.exp(ls[hh] - mx)
        es.append(e)
        ss.append(jnp.sum(e, axis=1, keepdims=True))
    outs = []
    for hh in range(h):
        vh = qkv[:, 2 * c + hh * c_:2 * c + (hh + 1) * c_]
        ov = jnp.dot(es[hh], vh, preferred_element_type=jnp.float32)
        outs.append(ov / ss[hh])
    o = jnp.concatenate(outs, axis=1).astype(jnp.bfloat16)
    out_ref[...] = jnp.dot(o, wproj_ref[...],
                           preferred_element_type=jnp.float32) + bproj_ref[...]


def _attn_tc(gf, ga, wqkv_t, bqkv, wpos, wproj_t, bproj, c, c_, h, d, m, zb):
    rows = gf.shape[0]
    aux_w = ga.shape[1]
    r = zb * m
    grid = (rows // r,)
    scale = c_ ** -0.5
    body = functools.partial(_attn_body, c, c_, h, d, m, zb, scale)
    return pl.pallas_call(
        body,
        grid=grid,
        in_specs=[
            pl.BlockSpec((r, c), lambda i: (i, 0)),
            pl.BlockSpec((r, aux_w), lambda i: (i, 0)),
            pl.BlockSpec((c, 3 * c), lambda i: (0, 0)),
            pl.BlockSpec((1, 3 * c), lambda i: (0, 0)),
            pl.BlockSpec((h, d), lambda i: (0, 0)),
            pl.BlockSpec((c, c), lambda i: (0, 0)),
            pl.BlockSpec((1, c), lambda i: (0, 0)),
        ],
        out_specs=pl.BlockSpec((r, c), lambda i: (i, 0)),
        out_shape=jax.ShapeDtypeStruct((rows, c), jnp.float32),
        compiler_params=pltpu.CompilerParams(
            dimension_semantics=("parallel",)),
    )(gf, ga, wqkv_t, bqkv, wpos, wproj_t, bproj)


def kernel(pos, feat, cluster_feat, mask, member_idx, batch_idx, k,
           valid_row_idx, attend_means, Wqkv, bqkv, Wpos, bpos, Wproj, bproj):
    b, n, c = feat.shape
    d = pos.shape[2]
    c_ = cluster_feat.shape[2]
    h = c // c_
    z, m = member_idx.shape

    idx = (batch_idx.astype(jnp.int32) * n
           + member_idx.astype(jnp.int32)).reshape(-1)
    aux_w = 128
    aux = jnp.concatenate(
        [cluster_feat, pos, mask.astype(jnp.float32),
         jnp.zeros((b, n, aux_w - c_ - d - 1), jnp.float32)],
        axis=-1).reshape(b * n, aux_w)

    feat_i = lax.bitcast_convert_type(
        feat.astype(jnp.bfloat16).reshape(b * n, c // 2, 2), jnp.int32)
    gf_i, ga = _sc_gather(feat_i, aux, idx)
    gf = lax.bitcast_convert_type(gf_i, jnp.bfloat16).reshape(z * m, c)

    zb = 2
    out = _attn_tc(gf, ga, Wqkv.T.astype(jnp.bfloat16),
                   bqkv.reshape(1, 3 * c), Wpos,
                   Wproj.T.astype(jnp.bfloat16), bproj.reshape(1, c),
                   c, c_, h, d, m, zb)
    return out.reshape(z, m, c)


# speedup vs baseline: 2.1087x; 2.1087x over previous
import functools

import jax
import jax.numpy as jnp
from jax import lax
from jax.experimental import pallas as pl
from jax.experimental.pallas import tpu as pltpu
from jax.experimental.pallas import tpu_sc as plsc

_NC = 2
_NS = 16
_NW = _NC * _NS
_CHUNK = 128


def _sc_gather(feat2d, aux2d, idx):
    rows, cf_w = feat2d.shape
    aux_w = aux2d.shape[1]
    per_w = rows // _NW
    n_chunks = per_w // _CHUNK
    mesh = plsc.VectorSubcoreMesh(core_axis_name="c", subcore_axis_name="s")

    @functools.partial(
        pl.kernel,
        mesh=mesh,
        out_type=(
            jax.ShapeDtypeStruct((rows, cf_w), jnp.float32),
            jax.ShapeDtypeStruct((rows, aux_w), jnp.float32),
        ),
        scratch_types=[
            pltpu.VMEM((_CHUNK,), jnp.int32),
            pltpu.VMEM((_CHUNK, cf_w), jnp.float32),
            pltpu.VMEM((_CHUNK, aux_w), jnp.float32),
            pltpu.SemaphoreType.DMA,
            pltpu.SemaphoreType.DMA,
        ],
    )
    def gather_kernel(feat_hbm, aux_hbm, idx_hbm, gf_hbm, ga_hbm,
                      idx_v, frows_v, arows_v, sem_f, sem_a):
        wid = lax.axis_index("s") * _NC + lax.axis_index("c")
        base = wid * per_w
        for ci in range(n_chunks):
            off = base + ci * _CHUNK
            pltpu.sync_copy(idx_hbm.at[pl.ds(off, _CHUNK)], idx_v)
            cp_f = pltpu.async_copy(feat_hbm.at[idx_v], frows_v, sem_f)
            cp_a = pltpu.async_copy(aux_hbm.at[idx_v], arows_v, sem_a)
            cp_f.wait()
            cp_a.wait()
            pltpu.sync_copy(frows_v, gf_hbm.at[pl.ds(off, _CHUNK)])
            pltpu.sync_copy(arows_v, ga_hbm.at[pl.ds(off, _CHUNK)])

    return gather_kernel(feat2d, aux2d, idx)


def _attn_body(c, c_, h, d, m, zb, scale,
               gf_ref, ga_ref, wqkv_ref, bqkv_ref, wpos_ref, wproj_ref,
               bproj_ref, out_ref):
    r = zb * m
    qkv = jnp.dot(gf_ref[...].astype(jnp.bfloat16), wqkv_ref[...],
                  preferred_element_type=jnp.float32) + bqkv_ref[...]
    aux_t = ga_ref[...].T
    cf = ga_ref[...][:, 0:c_]
    cf_t = aux_t[0:c_, :]
    pos_t = aux_t[c_:c_ + d, :]
    mask_t = aux_t[c_ + d:c_ + d + 1, :]
    sq_t = jnp.sum(cf_t * cf_t, axis=0, keepdims=True)
    pb_t = jnp.dot(wpos_ref[...], pos_t, precision=lax.Precision.HIGHEST,
                   preferred_element_type=jnp.float32)
    row_c = lax.broadcasted_iota(jnp.int32, (r, r), 0)
    col_c = lax.broadcasted_iota(jnp.int32, (r, r), 1)
    blockpen = jnp.where((row_c // m) == (col_c // m), 0.0, -1e30)
    cfdot = lax.dot_general(cf, cf, (((1,), (1,)), ((), ())),
                            precision=lax.Precision.HIGHEST,
                            preferred_element_type=jnp.float32)
    base = 2.0 * cfdot + blockpen + (-sq_t - 100.0 * (1.0 - mask_t))
    ls = []
    for hh in range(h):
        qh = qkv[:, hh * c_:(hh + 1) * c_] * scale
        kh = qkv[:, c + hh * c_:c + (hh + 1) * c_]
        l = lax.dot_general(qh, kh, (((1,), (1,)), ((), ())),
                            preferred_element_type=jnp.float32)
        ls.append(l + base + pb_t[hh:hh + 1, :])
    es, ss = [], []
    for hh in range(h):
        mx = jnp.max(ls[hh], axis=1, keepdims=True)
        e = jnp.exp(ls[hh] - mx)
        es.append(e)
        ss.append(jnp.sum(e, axis=1, keepdims=True))
    outs = []
    for hh in range(h):
        vh = qkv[:, 2 * c + hh * c_:2 * c + (hh + 1) * c_]
        ov = jnp.dot(es[hh], vh, preferred_element_type=jnp.float32)
        outs.append(ov / ss[hh])
    o = jnp.concatenate(outs, axis=1).astype(jnp.bfloat16)
    out_ref[...] = jnp.dot(o, wproj_ref[...],
                           preferred_element_type=jnp.float32) + bproj_ref[...]


def _attn_tc(gf, ga, wqkv_t, bqkv, wpos, wproj_t, bproj, c, c_, h, d, m, zb):
    rows = gf.shape[0]
    aux_w = ga.shape[1]
    r = zb * m
    grid = (rows // r,)
    scale = c_ ** -0.5
    body = functools.partial(_attn_body, c, c_, h, d, m, zb, scale)
    return pl.pallas_call(
        body,
        grid=grid,
        in_specs=[
            pl.BlockSpec((r, c), lambda i: (i, 0)),
            pl.BlockSpec((r, aux_w), lambda i: (i, 0)),
            pl.BlockSpec((c, 3 * c), lambda i: (0, 0)),
            pl.BlockSpec((1, 3 * c), lambda i: (0, 0)),
            pl.BlockSpec((h, d), lambda i: (0, 0)),
            pl.BlockSpec((c, c), lambda i: (0, 0)),
            pl.BlockSpec((1, c), lambda i: (0, 0)),
        ],
        out_specs=pl.BlockSpec((r, c), lambda i: (i, 0)),
        out_shape=jax.ShapeDtypeStruct((rows, c), jnp.float32),
        compiler_params=pltpu.CompilerParams(
            dimension_semantics=("parallel",)),
    )(gf, ga, wqkv_t, bqkv, wpos, wproj_t, bproj)


def kernel(pos, feat, cluster_feat, mask, member_idx, batch_idx, k,
           valid_row_idx, attend_means, Wqkv, bqkv, Wpos, bpos, Wproj, bproj):
    b, n, c = feat.shape
    d = pos.shape[2]
    c_ = cluster_feat.shape[2]
    h = c // c_
    z, m = member_idx.shape

    idx = (batch_idx.astype(jnp.int32) * n
           + member_idx.astype(jnp.int32)).reshape(-1)
    aux_w = 128
    aux = jnp.concatenate(
        [cluster_feat, pos, mask.astype(jnp.float32),
         jnp.zeros((b, n, aux_w - c_ - d - 1), jnp.float32)],
        axis=-1).reshape(b * n, aux_w)

    gf, ga = _sc_gather(feat.reshape(b * n, c), aux, idx)

    zb = 2
    out = _attn_tc(gf, ga, Wqkv.T.astype(jnp.bfloat16),
                   bqkv.reshape(1, 3 * c), Wpos,
                   Wproj.T.astype(jnp.bfloat16), bproj.reshape(1, c),
                   c, c_, h, d, m, zb)
    return out.reshape(z, m, c)
